# baseline (speedup 1.0000x reference)
import functools
import jax
import jax.numpy as jnp
from jax import lax
from jax.experimental import pallas as pl
from jax.experimental.pallas import tpu as pltpu
from jax.experimental.pallas import tpu_sc as plsc

D = 128
R = 512
NP = 10240
NC = 1
NS = 16
NW = NC * NS
CH = 64
SUP = 8
ACC_R = 10112
DROWS = NP // 128


def _tc_transform(x, w, a_src, a_dst, n_valid):
    def body(x_ref, w_ref, as_ref, ad_ref, h_ref, sv_ref, dv_ref, mx_ref):
        i = pl.program_id(0)
        h = jnp.dot(x_ref[...], w_ref[...], preferred_element_type=jnp.float32)
        h_ref[...] = h
        sv = jnp.sum(h * as_ref[0, :][None, :], axis=1)
        dv = jnp.sum(h * ad_ref[0, :][None, :], axis=1)
        sv_ref[...] = sv
        dv_ref[...] = dv

        @pl.when(i == 0)
        def _():
            mx_ref[...] = jnp.full((1, 128), -1e30, jnp.float32)

        rows = i * R + lax.broadcasted_iota(jnp.int32, (R,), 0)
        valid = rows < n_valid
        svm = jnp.max(jnp.where(valid, sv, -1e30))
        dvm = jnp.max(jnp.where(valid, dv, -1e30))
        lanes = lax.broadcasted_iota(jnp.int32, (1, 128), 1)
        upd = jnp.where(lanes == 0, svm, jnp.where(lanes == 1, dvm, -1e30))
        mx_ref[...] = jnp.maximum(mx_ref[...], upd)

    return pl.pallas_call(
        body,
        grid=(NP // R,),
        in_specs=[
            pl.BlockSpec((R, D), lambda i: (i, 0)),
            pl.BlockSpec((D, D), lambda i: (0, 0)),
            pl.BlockSpec((1, D), lambda i: (0, 0)),
            pl.BlockSpec((1, D), lambda i: (0, 0)),
        ],
        out_specs=[
            pl.BlockSpec((R, D), lambda i: (i, 0)),
            pl.BlockSpec((R,), lambda i: (i,)),
            pl.BlockSpec((R,), lambda i: (i,)),
            pl.BlockSpec((1, 128), lambda i: (0, 0)),
        ],
        out_shape=[
            jax.ShapeDtypeStruct((NP, D), jnp.float32),
            jax.ShapeDtypeStruct((NP,), jnp.float32),
            jax.ShapeDtypeStruct((NP,), jnp.float32),
            jax.ShapeDtypeStruct((1, 128), jnp.float32),
        ],
    )(x, w, a_src.reshape(1, D), a_dst.reshape(1, D))


def _tc_combine_body(nf_ref, den_ref, b_ref, fl_ref, out_ref):
    n = nf_ref[0]
    den = den_ref[0, :]
    for c in range(1, NC):
        n = n + nf_ref[c]
        den = den + den_ref[c, :]
    y = n / (den[:, None] + 1e-16) + b_ref[0, :][None, :]
    out_ref[...] = jnp.where(fl_ref[...] > 0.0, jnp.maximum(y, 0.0), y)


def _tc_combine(nf, den, b, flag_row):
    return pl.pallas_call(
        _tc_combine_body,
        grid=(NP // R,),
        in_specs=[
            pl.BlockSpec((NC, R, D), lambda i: (0, i, 0)),
            pl.BlockSpec((NC, R), lambda i: (0, i)),
            pl.BlockSpec((1, D), lambda i: (0, 0)),
            pl.BlockSpec((1, D), lambda i: (0, 0)),
        ],
        out_specs=pl.BlockSpec((R, D), lambda i: (i, 0)),
        out_shape=jax.ShapeDtypeStruct((NP, D), jnp.float32),
    )(nf, den, b.reshape(1, D), flag_row)


_SC_KERNEL_CACHE = {}


def _sc_edge_pass(h, src2d, dst2d, sv, dv, mx, n_chunks, e_total):
    key = (n_chunks, e_total)
    if key not in _SC_KERNEL_CACHE:
        _SC_KERNEL_CACHE[key] = _build_sc_edge_pass(n_chunks, e_total)
    return _SC_KERNEL_CACHE[key](h, src2d, dst2d, sv, dv, mx)


def _build_sc_edge_pass(n_chunks, e_total):
    rows_per_tile = ACC_R // NS
    mesh = plsc.VectorSubcoreMesh(core_axis_name="c", subcore_axis_name="s",
                                  num_cores=NC)

    n_sup = n_chunks // SUP

    @functools.partial(
        pl.kernel,
        out_type=(
            jax.ShapeDtypeStruct((NC, ACC_R, D), jnp.float32),
            jax.ShapeDtypeStruct((NC, DROWS, 128), jnp.float32),
        ),
        mesh=mesh,
        scratch_types=[
            pltpu.VMEM((2, SUP, CH), jnp.int32),
            pltpu.VMEM((2, SUP, CH), jnp.int32),
            pltpu.VMEM((2, CH), jnp.float32),
            pltpu.VMEM((2, CH), jnp.float32),
            pltpu.VMEM((128,), jnp.float32),
            pltpu.VMEM((3, CH, D), jnp.float32),
            pltpu.VMEM((DROWS, 128), jnp.float32),
            pltpu.VMEM((DROWS,), jnp.int32),
            pltpu.VMEM_SHARED((ACC_R, D), jnp.float32),
            pltpu.VMEM_SHARED((DROWS, 128), jnp.float32),
            pltpu.SemaphoreType.DMA,
            pltpu.SemaphoreType.DMA,
            pltpu.SemaphoreType.DMA,
            pltpu.SemaphoreType.DMA,
        ],
        compiler_params=pltpu.CompilerParams(needs_layout_passes=False),
    )
    def body(h_hbm, src_hbm, dst_hbm, sv_hbm, dv_hbm, mx_hbm,
             out_hbm, den_hbm,
             src_l, dst_l, svb, dvb, mx_l, gbuf,
             den_l, rowid, accum, den_sh, isem, gsem, ssem, vsem):
        ci = lax.axis_index("c")
        si = lax.axis_index("s")
        wid = si * NC + ci
        tile_edges = n_chunks * CH
        ebase = wid * tile_edges
        row0 = si * rows_per_tile

        lanes = lax.iota(jnp.int32, 16)
        zeros16f = jnp.zeros((16,), jnp.float32)
        zero16 = jnp.zeros((16,), jnp.int32)

        def zrow(r, _):
            for j in range(D // 16):
                gbuf[0, r, pl.ds(j * 16, 16)] = zeros16f
            return 0

        lax.fori_loop(0, CH, zrow, 0)

        def zden(r, _):
            for j in range(128 // 16):
                den_l[r, pl.ds(j * 16, 16)] = zeros16f
            return 0

        lax.fori_loop(0, DROWS, zden, 0)

        def zrid(r, _):
            rowid[pl.ds(r * 16, 16)] = r * 16 + lanes
            return 0

        lax.fori_loop(0, DROWS // 16, zrid, 0)

        n_rb = -(-rows_per_tile // 32)

        def zacc(z, _):
            off = jnp.minimum(z * 32, rows_per_tile - 32)
            pltpu.sync_copy(gbuf.at[0, pl.ds(0, 32)],
                            accum.at[pl.ds(row0 + off, 32)])
            return 0

        lax.fori_loop(0, n_rb, zacc, 0)

        @pl.when(si == 0)
        def _():
            pltpu.sync_copy(den_l, den_sh)

        pltpu.sync_copy(mx_hbm.at[0], mx_l)

        mxv = mx_l[pl.ds(0, 16)]
        c0 = mxv[0] + mxv[1]
        c = jnp.where(c0 >= 0.0, c0, 0.2 * c0)

        plsc.subcore_barrier()

        def issue_idx(s):
            sl = lax.rem(s, 2)
            pltpu.async_copy(src_hbm.at[wid, pl.ds(s * SUP, SUP)],
                             src_l.at[sl], isem)
            pltpu.async_copy(dst_hbm.at[wid, pl.ds(s * SUP, SUP)],
                             dst_l.at[sl], isem)

        def wait_idx(s):
            sl = lax.rem(s, 2)
            pltpu.make_async_copy(src_hbm.at[wid, pl.ds(s * SUP, SUP)],
                                  src_l.at[sl], isem).wait()
            pltpu.make_async_copy(dst_hbm.at[wid, pl.ds(s * SUP, SUP)],
                                  dst_l.at[sl], isem).wait()

        def issue_gather(g):
            sl = lax.rem(g, 3)
            vsl = lax.rem(g, 2)
            isl = lax.rem(lax.div(g, SUP), 2)
            goff = lax.rem(g, SUP)
            pltpu.async_copy(h_hbm.at[src_l.at[isl, goff]],
                             gbuf.at[sl], gsem)
            pltpu.async_copy(sv_hbm.at[src_l.at[isl, goff]],
                             svb.at[vsl], vsem)
            pltpu.async_copy(dv_hbm.at[dst_l.at[isl, goff]],
                             dvb.at[vsl], vsem)

        def wait_logits():
            pltpu.make_async_copy(sv_hbm.at[pl.ds(0, CH)], svb.at[0],
                                  vsem).wait()
            pltpu.make_async_copy(dv_hbm.at[pl.ds(0, CH)], dvb.at[0],
                                  vsem).wait()

        def wait_gather(g):
            sl = lax.rem(g, 3)
            pltpu.make_async_copy(h_hbm.at[pl.ds(0, CH)], gbuf.at[sl],
                                  gsem).wait()

        def wait_scatter():
            pltpu.make_async_copy(h_hbm.at[pl.ds(0, CH)], gbuf.at[0],
                                  ssem).wait()

        issue_idx(0)
        wait_idx(0)
        issue_gather(0)

        def chunk(g, _):
            isl = lax.rem(lax.div(g, SUP), 2)
            gsl = lax.rem(g, 3)
            goff = lax.rem(g, SUP)

            @pl.when(g >= 2)
            def _():
                wait_scatter()

            @pl.when(jnp.logical_and(lax.rem(g, SUP) == 0,
                                     lax.div(g, SUP) + 1 < n_sup))
            def _():
                issue_idx(lax.div(g, SUP) + 1)

            @pl.when(jnp.logical_and(lax.rem(g + 1, SUP) == 0,
                                     g + 1 < n_chunks))
            def _():
                wait_idx(lax.div(g + 1, SUP))

            @pl.when(g + 1 < n_chunks)
            def _():
                issue_gather(g + 1)

            wait_logits()
            vsl = lax.rem(g, 2)
            exs = []
            for j in range(CH // 16):
                d16 = dst_l[isl, goff, pl.ds(j * 16, 16)]
                e = svb[vsl, pl.ds(j * 16, 16)] + dvb[vsl, pl.ds(j * 16, 16)]
                e = jnp.where(e >= 0.0, e, 0.2 * e) - c
                ex = jnp.exp(e)
                gid = ebase + g * CH + j * 16 + lanes
                ex = jnp.where(gid < e_total, ex, 0.0)
                exs.append(ex)
                plsc.addupdate_scatter(
                    den_l,
                    [lax.shift_right_arithmetic(d16, 7),
                     lax.bitwise_and(d16, 127)],
                    ex)

            wait_gather(g)

            for j in range(CH // 16):
                ex16 = exs[j]
                for i in range(16):
                    s = ex16[i]
                    row = j * 16 + i
                    for k in range(D // 16):
                        gbuf[gsl, row, pl.ds(k * 16, 16)] = (
                            gbuf[gsl, row, pl.ds(k * 16, 16)] * s)

            pltpu.async_copy(gbuf.at[gsl], accum.at[dst_l.at[isl, goff]],
                             ssem, add=True)
            return 0

        lax.fori_loop(0, n_chunks, chunk, 0)
        wait_scatter()
        wait_scatter()

        pltpu.sync_copy(den_l, den_sh.at[rowid], add=True)
        plsc.subcore_barrier()

        def rback(z, _):
            off = jnp.minimum(z * 32, rows_per_tile - 32)
            pltpu.sync_copy(accum.at[pl.ds(row0 + off, 32)],
                            gbuf.at[0, pl.ds(0, 32)])
            pltpu.sync_copy(gbuf.at[0, pl.ds(0, 32)],
                            out_hbm.at[ci, pl.ds(row0 + off, 32)])
            return 0

        lax.fori_loop(0, n_rb, rback, 0)

        @pl.when(si < DROWS // 8)
        def _():
            pltpu.sync_copy(den_sh.at[pl.ds(si * 8, 8)], gbuf.at[0, pl.ds(0, 8)])
            pltpu.sync_copy(gbuf.at[0, pl.ds(0, 8)],
                            den_hbm.at[ci, pl.ds(si * 8, 8)])

    return body


def kernel(x, edge_index, W1, att_src1, att_dst1, b1,
           W2, att_src2, att_dst2, b2):
    n, d = x.shape
    assert d == D and n <= NP
    e = edge_index.shape[1]
    e_total = e + n
    n_chunks = -(-e_total // (NW * CH))
    n_chunks = -(-n_chunks // SUP) * SUP
    ep = n_chunks * NW * CH

    loops = jnp.arange(n, dtype=jnp.int32)
    pad = jnp.zeros((ep - e_total,), jnp.int32)
    src = jnp.concatenate([edge_index[0].astype(jnp.int32), loops, pad])
    dst = jnp.concatenate([edge_index[1].astype(jnp.int32), loops, pad])
    src2d = src.reshape(NW, n_chunks, CH)
    dst2d = dst.reshape(NW, n_chunks, CH)
    xp = jnp.pad(x, ((0, NP - n), (0, 0)))

    ws = jnp.stack([W1, W2])
    avs = jnp.stack([att_src1, att_src2])
    ads = jnp.stack([att_dst1, att_dst2])
    bs = jnp.stack([b1, b2])

    def layer(i, hcur):
        w = lax.dynamic_index_in_dim(ws, i, keepdims=False)
        av = lax.dynamic_index_in_dim(avs, i, keepdims=False)
        ad = lax.dynamic_index_in_dim(ads, i, keepdims=False)
        b = lax.dynamic_index_in_dim(bs, i, keepdims=False)
        hh, sv, dv, mx = _tc_transform(hcur, w, av, ad, n)
        nf, dn = _sc_edge_pass(hh, src2d, dst2d, sv, dv, mx,
                               n_chunks, e_total)
        flag = jnp.where(i == 0, 1.0, 0.0).astype(jnp.float32)
        flag_row = jnp.full((1, D), flag, jnp.float32)
        return _tc_combine(nf, dn.reshape(NC, NP), b, flag_row)

    nlayers = 2 + jnp.minimum(edge_index.ravel()[0] * 0, 0)
    out = lax.fori_loop(0, nlayers, layer, xp)
    return out[:n]


# speedup vs baseline: 1.2732x; 1.2732x over previous
import functools
import jax
import jax.numpy as jnp
from jax import lax
from jax.experimental import pallas as pl
from jax.experimental.pallas import tpu as pltpu
from jax.experimental.pallas import tpu_sc as plsc

D = 128
R = 512
NP = 10240
NC = 1
NS = 16
NW = NC * NS
CH = 32
SUP = 8
ACC_R = 10112
DROWS = NP // 128


def _tc_transform(x, w, a_src, a_dst, n_valid):
    def body(x_ref, w_ref, as_ref, ad_ref, h_ref, sv_ref, dv_ref, mx_ref):
        i = pl.program_id(0)
        h = jnp.dot(x_ref[...], w_ref[...], preferred_element_type=jnp.float32)
        h_ref[...] = h
        sv = jnp.sum(h * as_ref[0, :][None, :], axis=1)
        dv = jnp.sum(h * ad_ref[0, :][None, :], axis=1)
        sv_ref[...] = sv
        dv_ref[...] = dv

        @pl.when(i == 0)
        def _():
            mx_ref[...] = jnp.full((1, 128), -1e30, jnp.float32)

        rows = i * R + lax.broadcasted_iota(jnp.int32, (R,), 0)
        valid = rows < n_valid
        svm = jnp.max(jnp.where(valid, sv, -1e30))
        dvm = jnp.max(jnp.where(valid, dv, -1e30))
        lanes = lax.broadcasted_iota(jnp.int32, (1, 128), 1)
        upd = jnp.where(lanes == 0, svm, jnp.where(lanes == 1, dvm, -1e30))
        mx_ref[...] = jnp.maximum(mx_ref[...], upd)

    return pl.pallas_call(
        body,
        grid=(NP // R,),
        in_specs=[
            pl.BlockSpec((R, D), lambda i: (i, 0)),
            pl.BlockSpec((D, D), lambda i: (0, 0)),
            pl.BlockSpec((1, D), lambda i: (0, 0)),
            pl.BlockSpec((1, D), lambda i: (0, 0)),
        ],
        out_specs=[
            pl.BlockSpec((R, D), lambda i: (i, 0)),
            pl.BlockSpec((R,), lambda i: (i,)),
            pl.BlockSpec((R,), lambda i: (i,)),
            pl.BlockSpec((1, 128), lambda i: (0, 0)),
        ],
        out_shape=[
            jax.ShapeDtypeStruct((NP, D), jnp.float32),
            jax.ShapeDtypeStruct((NP,), jnp.float32),
            jax.ShapeDtypeStruct((NP,), jnp.float32),
            jax.ShapeDtypeStruct((1, 128), jnp.float32),
        ],
    )(x, w, a_src.reshape(1, D), a_dst.reshape(1, D))


def _tc_combine_body(nf_ref, den_ref, b_ref, fl_ref, out_ref):
    n = nf_ref[0]
    den = den_ref[0, :]
    for c in range(1, NC):
        n = n + nf_ref[c]
        den = den + den_ref[c, :]
    y = n / (den[:, None] + 1e-16) + b_ref[0, :][None, :]
    out_ref[...] = jnp.where(fl_ref[...] > 0.0, jnp.maximum(y, 0.0), y)


def _tc_combine(nf, den, b, flag_row):
    return pl.pallas_call(
        _tc_combine_body,
        grid=(NP // R,),
        in_specs=[
            pl.BlockSpec((NC, R, D), lambda i: (0, i, 0)),
            pl.BlockSpec((NC, R), lambda i: (0, i)),
            pl.BlockSpec((1, D), lambda i: (0, 0)),
            pl.BlockSpec((1, D), lambda i: (0, 0)),
        ],
        out_specs=pl.BlockSpec((R, D), lambda i: (i, 0)),
        out_shape=jax.ShapeDtypeStruct((NP, D), jnp.float32),
    )(nf, den, b.reshape(1, D), flag_row)


_SC_KERNEL_CACHE = {}


def _sc_edge_pass(h, src2d, dst2d, sv, dv, mx, n_chunks, e_total):
    key = (n_chunks, e_total)
    if key not in _SC_KERNEL_CACHE:
        _SC_KERNEL_CACHE[key] = _build_sc_edge_pass(n_chunks, e_total)
    return _SC_KERNEL_CACHE[key](h, src2d, dst2d, sv, dv, mx)


def _build_sc_edge_pass(n_chunks, e_total):
    rows_per_tile = ACC_R // NS
    mesh = plsc.VectorSubcoreMesh(core_axis_name="c", subcore_axis_name="s",
                                  num_cores=NC)

    n_sup = n_chunks // SUP

    @functools.partial(
        pl.kernel,
        out_type=(
            jax.ShapeDtypeStruct((NC, ACC_R, D), jnp.float32),
            jax.ShapeDtypeStruct((NC, DROWS, 128), jnp.float32),
        ),
        mesh=mesh,
        scratch_types=[
            pltpu.VMEM((2, SUP, CH), jnp.int32),
            pltpu.VMEM((2, SUP, CH), jnp.int32),
            pltpu.VMEM((NP,), jnp.float32),
            pltpu.VMEM((NP,), jnp.float32),
            pltpu.VMEM((128,), jnp.float32),
            pltpu.VMEM((3, CH, D), jnp.float32),
            pltpu.VMEM((DROWS, 128), jnp.float32),
            pltpu.VMEM((DROWS,), jnp.int32),
            pltpu.VMEM_SHARED((ACC_R, D), jnp.float32),
            pltpu.VMEM_SHARED((DROWS, 128), jnp.float32),
            pltpu.SemaphoreType.DMA,
            pltpu.SemaphoreType.DMA,
            pltpu.SemaphoreType.DMA,
            pltpu.SemaphoreType.DMA,
        ],
        compiler_params=pltpu.CompilerParams(needs_layout_passes=False),
    )
    def body(h_hbm, src_hbm, dst_hbm, sv_hbm, dv_hbm, mx_hbm,
             out_hbm, den_hbm,
             src_l, dst_l, sv_l, dv_l, mx_l, gbuf,
             den_l, rowid, accum, den_sh, isem, gsem, ssem, vsem):
        ci = lax.axis_index("c")
        si = lax.axis_index("s")
        wid = si * NC + ci
        tile_edges = n_chunks * CH
        ebase = wid * tile_edges
        row0 = si * rows_per_tile

        lanes = lax.iota(jnp.int32, 16)
        zeros16f = jnp.zeros((16,), jnp.float32)
        zero16 = jnp.zeros((16,), jnp.int32)

        def zrow(r, _):
            for j in range(D // 16):
                gbuf[0, r, pl.ds(j * 16, 16)] = zeros16f
            return 0

        lax.fori_loop(0, CH, zrow, 0)

        def zden(r, _):
            for j in range(128 // 16):
                den_l[r, pl.ds(j * 16, 16)] = zeros16f
            return 0

        lax.fori_loop(0, DROWS, zden, 0)

        def zrid(r, _):
            rowid[pl.ds(r * 16, 16)] = r * 16 + lanes
            return 0

        lax.fori_loop(0, DROWS // 16, zrid, 0)

        n_rb = -(-rows_per_tile // 32)

        def zacc(z, _):
            off = jnp.minimum(z * 32, rows_per_tile - 32)
            pltpu.sync_copy(gbuf.at[0, pl.ds(0, 32)],
                            accum.at[pl.ds(row0 + off, 32)])
            return 0

        lax.fori_loop(0, n_rb, zacc, 0)

        @pl.when(si == 0)
        def _():
            pltpu.sync_copy(den_l, den_sh)

        pltpu.sync_copy(sv_hbm, sv_l)
        pltpu.sync_copy(dv_hbm, dv_l)
        pltpu.sync_copy(mx_hbm.at[0], mx_l)

        mxv = mx_l[pl.ds(0, 16)]
        c0 = mxv[0] + mxv[1]
        c = jnp.where(c0 >= 0.0, c0, 0.2 * c0)

        plsc.subcore_barrier()

        def issue_idx(s):
            sl = lax.rem(s, 2)
            pltpu.async_copy(src_hbm.at[wid, pl.ds(s * SUP, SUP)],
                             src_l.at[sl], isem)
            pltpu.async_copy(dst_hbm.at[wid, pl.ds(s * SUP, SUP)],
                             dst_l.at[sl], isem)

        def wait_idx(s):
            sl = lax.rem(s, 2)
            pltpu.make_async_copy(src_hbm.at[wid, pl.ds(s * SUP, SUP)],
                                  src_l.at[sl], isem).wait()
            pltpu.make_async_copy(dst_hbm.at[wid, pl.ds(s * SUP, SUP)],
                                  dst_l.at[sl], isem).wait()

        def issue_gather(g):
            sl = lax.rem(g, 3)
            isl = lax.rem(lax.div(g, SUP), 2)
            goff = lax.rem(g, SUP)
            pltpu.async_copy(h_hbm.at[src_l.at[isl, goff]],
                             gbuf.at[sl], gsem)

        def wait_gather(g):
            sl = lax.rem(g, 3)
            pltpu.make_async_copy(h_hbm.at[pl.ds(0, CH)], gbuf.at[sl],
                                  gsem).wait()

        def wait_scatter():
            pltpu.make_async_copy(h_hbm.at[pl.ds(0, CH)], gbuf.at[0],
                                  ssem).wait()

        issue_idx(0)
        wait_idx(0)
        issue_gather(0)
        issue_gather(1)

        def chunk(g, _):
            isl = lax.rem(lax.div(g, SUP), 2)
            gsl = lax.rem(g, 3)
            goff = lax.rem(g, SUP)

            @pl.when(jnp.logical_and(lax.rem(g, SUP) == 0,
                                     lax.div(g, SUP) + 1 < n_sup))
            def _():
                issue_idx(lax.div(g, SUP) + 1)

            @pl.when(jnp.logical_and(lax.rem(g + 2, SUP) == 0,
                                     g + 2 < n_chunks))
            def _():
                wait_idx(lax.div(g + 2, SUP))

            exs = []
            for j in range(CH // 16):
                s16 = src_l[isl, goff, pl.ds(j * 16, 16)]
                d16 = dst_l[isl, goff, pl.ds(j * 16, 16)]
                e = plsc.load_gather(sv_l, [s16]) + plsc.load_gather(dv_l, [d16])
                e = jnp.where(e >= 0.0, e, 0.2 * e) - c
                ex = jnp.exp(e)
                gid = ebase + g * CH + j * 16 + lanes
                ex = jnp.where(gid < e_total, ex, 0.0)
                exs.append(ex)
                plsc.addupdate_scatter(
                    den_l,
                    [lax.shift_right_arithmetic(d16, 7),
                     lax.bitwise_and(d16, 127)],
                    ex)

            @pl.when(g >= 1)
            def _():
                wait_scatter()

            @pl.when(g + 2 < n_chunks)
            def _():
                issue_gather(g + 2)

            wait_gather(g)

            for j in range(CH // 16):
                ex16 = exs[j]
                for i in range(16):
                    s = ex16[i]
                    row = j * 16 + i
                    for k in range(D // 16):
                        gbuf[gsl, row, pl.ds(k * 16, 16)] = (
                            gbuf[gsl, row, pl.ds(k * 16, 16)] * s)

            pltpu.async_copy(gbuf.at[gsl], accum.at[dst_l.at[isl, goff]],
                             ssem, add=True)
            return 0

        lax.fori_loop(0, n_chunks, chunk, 0)
        wait_scatter()

        pltpu.sync_copy(den_l, den_sh.at[rowid], add=True)
        plsc.subcore_barrier()

        def rback(z, _):
            off = jnp.minimum(z * 32, rows_per_tile - 32)
            pltpu.sync_copy(accum.at[pl.ds(row0 + off, 32)],
                            gbuf.at[0, pl.ds(0, 32)])
            pltpu.sync_copy(gbuf.at[0, pl.ds(0, 32)],
                            out_hbm.at[ci, pl.ds(row0 + off, 32)])
            return 0

        lax.fori_loop(0, n_rb, rback, 0)

        @pl.when(si < DROWS // 8)
        def _():
            pltpu.sync_copy(den_sh.at[pl.ds(si * 8, 8)], gbuf.at[0, pl.ds(0, 8)])
            pltpu.sync_copy(gbuf.at[0, pl.ds(0, 8)],
                            den_hbm.at[ci, pl.ds(si * 8, 8)])

    return body


def kernel(x, edge_index, W1, att_src1, att_dst1, b1,
           W2, att_src2, att_dst2, b2):
    n, d = x.shape
    assert d == D and n <= NP
    e = edge_index.shape[1]
    e_total = e + n
    n_chunks = -(-e_total // (NW * CH))
    n_chunks = -(-n_chunks // SUP) * SUP
    ep = n_chunks * NW * CH

    loops = jnp.arange(n, dtype=jnp.int32)
    pad = jnp.zeros((ep - e_total,), jnp.int32)
    src = jnp.concatenate([edge_index[0].astype(jnp.int32), loops, pad])
    dst = jnp.concatenate([edge_index[1].astype(jnp.int32), loops, pad])
    src2d = src.reshape(NW, n_chunks, CH)
    dst2d = dst.reshape(NW, n_chunks, CH)
    xp = jnp.pad(x, ((0, NP - n), (0, 0)))

    ws = jnp.stack([W1, W2])
    avs = jnp.stack([att_src1, att_src2])
    ads = jnp.stack([att_dst1, att_dst2])
    bs = jnp.stack([b1, b2])

    def layer(i, hcur):
        w = lax.dynamic_index_in_dim(ws, i, keepdims=False)
        av = lax.dynamic_index_in_dim(avs, i, keepdims=False)
        ad = lax.dynamic_index_in_dim(ads, i, keepdims=False)
        b = lax.dynamic_index_in_dim(bs, i, keepdims=False)
        hh, sv, dv, mx = _tc_transform(hcur, w, av, ad, n)
        nf, dn = _sc_edge_pass(hh, src2d, dst2d, sv, dv, mx,
                               n_chunks, e_total)
        flag = jnp.where(i == 0, 1.0, 0.0).astype(jnp.float32)
        flag_row = jnp.full((1, D), flag, jnp.float32)
        return _tc_combine(nf, dn.reshape(NC, NP), b, flag_row)

    nlayers = 2 + jnp.minimum(edge_index.ravel()[0] * 0, 0)
    out = lax.fori_loop(0, nlayers, layer, xp)
    return out[:n]
